# baseline (speedup 1.0000x reference)
import functools

import jax
import jax.numpy as jnp
from jax import lax
from jax.experimental import pallas as pl
from jax.experimental.pallas import tpu as pltpu
from jax.experimental.pallas import tpu_sc as plsc

N = 10000
M = 2500
M_PAD = 2560
K = 32
C_MID = 16
D = 128
D_OUT = 128
D_GLOBAL = 256
EPS = 1e-5

NROW = 80
NCOL = 10112
E = M * K
E_PAD = M_PAD * K
BIG_I = 2**30


def _fps_body(px_ref, py_ref, pz_ref, psx_ref, psy_ref, psz_ref, idx_ref):
    px = px_ref[...]
    py = py_ref[...]
    pz = pz_ref[...]
    flat = (lax.broadcasted_iota(jnp.int32, (NROW, 128), 0) * 128
            + lax.broadcasted_iota(jnp.int32, (NROW, 128), 1))
    valid = flat < N
    lane = lax.broadcasted_iota(jnp.int32, (NROW, 128), 1)
    rowbase = lax.broadcasted_iota(jnp.int32, (NROW, 1), 0) * 128

    def dist_to(j):
        lx = psx_ref[j]
        ly = psy_ref[j]
        lz = psz_ref[j]
        return (px - lx) ** 2 + (py - ly) ** 2 + (pz - lz) ** 2

    idx_ref[0] = jnp.int32(0)
    dmin0 = jnp.where(valid, dist_to(jnp.int32(0)), -jnp.inf)

    def argmax_flat(dmin):
        v, f = dmin, flat
        for rows in (40, 24, 16, 8):
            lo_v, hi_v = v[:rows], v[rows:]
            lo_f, hi_f = f[:rows], f[rows:]
            k = lo_v.shape[0] - hi_v.shape[0]
            if k:
                keep_v, keep_f = lo_v[:k], lo_f[:k]
                lo_v, lo_f = lo_v[k:], lo_f[k:]
            t = hi_v > lo_v
            nv = jnp.where(t, hi_v, lo_v)
            nf = jnp.where(t, hi_f, lo_f)
            v = jnp.concatenate([keep_v, nv]) if k else nv
            f = jnp.concatenate([keep_f, nf]) if k else nf
        for rows in (4, 2, 1):
            t = v[rows:] > v[:rows]
            v = jnp.where(t, v[rows:], v[:rows])
            f = jnp.where(t, f[rows:], f[:rows])
        for s in (64, 32, 16, 8, 4, 2, 1):
            v2 = pltpu.roll(v, s, axis=1)
            f2 = pltpu.roll(f, s, axis=1)
            t = (v2 > v) | ((v2 == v) & (f2 < f))
            v = jnp.where(t, v2, v)
            f = jnp.where(t, f2, f)
        return jnp.max(f[0:1, 0:1])

    def body(i, dmin):
        j = argmax_flat(dmin)
        idx_ref[i] = j
        return jnp.minimum(dmin, dist_to(j))

    lax.fori_loop(1, M, body, dmin0)


def _fps_call(px, py, pz, psx, psy, psz, interpret=False):
    return pl.pallas_call(
        _fps_body,
        out_shape=jax.ShapeDtypeStruct((M,), jnp.int32),
        in_specs=[pl.BlockSpec(memory_space=pltpu.VMEM)] * 3
        + [pl.BlockSpec(memory_space=pltpu.SMEM)] * 3,
        out_specs=pl.BlockSpec(memory_space=pltpu.SMEM),
        interpret=interpret,
    )(px, py, pz, psx, psy, psz)


KNN_T = 8
KNN_G = NCOL // 128


def _knn_body(spos_ref, pc_ref, nbr_ref):
    bm = spos_ref.shape[0]
    sx = spos_ref[:, 0:1]
    sy = spos_ref[:, 1:2]
    sz = spos_ref[:, 2:3]
    px = pc_ref[0:1, :]
    py = pc_ref[1:2, :]
    pz = pc_ref[2:3, :]
    d2 = (sx - px) ** 2 + (sy - py) ** 2 + (sz - pz) ** 2
    cidx = lax.broadcasted_iota(jnp.int32, d2.shape, 1)
    d2 = jnp.where(cidx < N, d2, jnp.inf)

    inf = jnp.inf
    tvals = [jnp.full((bm, 128), inf, jnp.float32) for _ in range(KNN_T)]
    targs = [jnp.zeros((bm, 128), jnp.int32) for _ in range(KNN_T)]
    for g in range(KNN_G):
        cv = d2[:, g * 128:(g + 1) * 128]
        ca = None
        for p in range(KNN_T):
            sw = cv < tvals[p]
            tv_new = jnp.where(sw, cv, tvals[p])
            cv = jnp.where(sw, tvals[p], cv)
            if ca is None:
                ta_new = jnp.where(sw, g, targs[p])
                ca = jnp.where(sw, targs[p], g)
            else:
                ta_new = jnp.where(sw, ca, targs[p])
                ca = jnp.where(sw, targs[p], ca)
            tvals[p] = tv_new
            targs[p] = ta_new

    lidx = lax.broadcasted_iota(jnp.int32, (bm, 128), 1)
    cur, curg = tvals[0], targs[0]
    cnt = jnp.zeros((bm, 128), jnp.int32)
    cols = []
    for _ in range(K):
        v = jnp.min(cur, axis=1, keepdims=True)
        l = jnp.min(jnp.where(cur == v, lidx, BIG_I), axis=1, keepdims=True)
        oh = lidx == l
        g = jnp.sum(jnp.where(oh, curg, 0), axis=1, keepdims=True)
        cols.append(g * 128 + l)
        newc = cnt + 1
        nv = jnp.full((bm, 128), inf, jnp.float32)
        ng = jnp.zeros((bm, 128), jnp.int32)
        for p in range(1, KNN_T):
            hit = newc == p
            nv = jnp.where(hit, tvals[p], nv)
            ng = jnp.where(hit, targs[p], ng)
        cur = jnp.where(oh, nv, cur)
        curg = jnp.where(oh, ng, curg)
        cnt = jnp.where(oh, newc, cnt)
    nbr_ref[...] = jnp.concatenate(cols, axis=1)


def _knn_call(spos16, pos_cols, interpret=False):
    bm = 128
    grid = M_PAD // bm
    return pl.pallas_call(
        _knn_body,
        grid=(grid,),
        in_specs=[
            pl.BlockSpec((bm, 16), lambda i: (i, 0)),
            pl.BlockSpec((16, NCOL), lambda i: (0, 0)),
        ],
        out_specs=pl.BlockSpec((bm, K), lambda i: (i, 0)),
        out_shape=jax.ShapeDtypeStruct((M_PAD, K), jnp.int32),
        interpret=interpret,
    )(spos16, pos_cols)


def _xw_body(x_ref, w_ref, o_ref):
    o_ref[...] = jnp.dot(x_ref[...], w_ref[...],
                         preferred_element_type=jnp.float32)


def _xw_call(x, w2, interpret=False):
    bm = 1000
    return pl.pallas_call(
        _xw_body,
        grid=(N // bm,),
        in_specs=[
            pl.BlockSpec((bm, D), lambda i: (i, 0)),
            pl.BlockSpec((D, D_OUT), lambda i: (0, 0)),
        ],
        out_specs=pl.BlockSpec((bm, D_OUT), lambda i: (i, 0)),
        out_shape=jax.ShapeDtypeStruct((N, D_OUT), jnp.float32),
        interpret=interpret,
    )(x, w2)


def _sc_gather(table, idx, chunk):
    b = idx.shape[0]
    d = table.shape[1]
    nw = 32
    bpw = b // nw
    nch = bpw // chunk
    mesh = plsc.VectorSubcoreMesh(core_axis_name="c", subcore_axis_name="s")

    @functools.partial(
        pl.kernel,
        mesh=mesh,
        out_type=jax.ShapeDtypeStruct((b, d), jnp.float32),
        scratch_types=[
            pltpu.VMEM((chunk,), jnp.int32),
            pltpu.VMEM((chunk, d), jnp.float32),
            pltpu.SemaphoreType.DMA,
        ],
    )
    def k(table_hbm, idx_hbm, out_hbm, idx_v, rows_v, sem):
        wid = lax.axis_index("s") * 2 + lax.axis_index("c")
        base = wid * bpw
        for c in range(nch):
            off = base + c * chunk
            pltpu.sync_copy(idx_hbm.at[pl.ds(off, chunk)], idx_v)
            pltpu.async_copy(table_hbm.at[idx_v], rows_v, sem).wait()
            pltpu.sync_copy(rows_v, out_hbm.at[pl.ds(off, chunk)])

    return k(table, idx)


def _p1_body(posj_ref, spos_ref, xwj_ref, wnw_ref, wnb_ref, w1_ref, lnb_ref,
             wpre_ref, nppre_ref, st_ref):
    i = pl.program_id(0)
    bm = spos_ref.shape[0]
    be = bm * K
    spos = spos_ref[...]
    spos_e = jnp.broadcast_to(spos[:, None, :], (bm, K, 16)).reshape(be, 16)
    gn3 = (posj_ref[...] - spos_e)[:, 0:3]
    w = jnp.maximum(
        jnp.dot(gn3, wnw_ref[...], preferred_element_type=jnp.float32)
        + wnb_ref[...], 0.0)
    npv = jnp.maximum(
        jnp.dot(gn3, w1_ref[...], preferred_element_type=jnp.float32)
        + xwj_ref[...] + lnb_ref[...], 0.0)
    wpre_ref[...] = w
    nppre_ref[...] = npv

    eidx = i * be + lax.broadcasted_iota(jnp.int32, (be, 1), 0)
    mask = eidx < E
    wm = jnp.where(mask, w, 0.0)
    nm = jnp.where(mask, npv, 0.0)

    @pl.when(i == 0)
    def _():
        st_ref[...] = jnp.zeros_like(st_ref)

    st_ref[0:1, 0:C_MID] += jnp.sum(wm, axis=0, keepdims=True)
    st_ref[1:2, 0:C_MID] += jnp.sum(wm * wm, axis=0, keepdims=True)
    st_ref[2:3, :] += jnp.sum(nm, axis=0, keepdims=True)
    st_ref[3:4, :] += jnp.sum(nm * nm, axis=0, keepdims=True)


def _p1_call(posj, spos16, xwj, wn_W, wn_b, W1, ln_b, interpret=False):
    bm = 32
    be = bm * K
    grid = M_PAD // bm
    return pl.pallas_call(
        _p1_body,
        grid=(grid,),
        in_specs=[
            pl.BlockSpec((be, 16), lambda i: (i, 0)),
            pl.BlockSpec((bm, 16), lambda i: (i, 0)),
            pl.BlockSpec((be, D_OUT), lambda i: (i, 0)),
            pl.BlockSpec((3, C_MID), lambda i: (0, 0)),
            pl.BlockSpec((1, C_MID), lambda i: (0, 0)),
            pl.BlockSpec((3, D_OUT), lambda i: (0, 0)),
            pl.BlockSpec((1, D_OUT), lambda i: (0, 0)),
        ],
        out_specs=[
            pl.BlockSpec((be, C_MID), lambda i: (i, 0)),
            pl.BlockSpec((be, D_OUT), lambda i: (i, 0)),
            pl.BlockSpec((8, 128), lambda i: (0, 0)),
        ],
        out_shape=[
            jax.ShapeDtypeStruct((E_PAD, C_MID), jnp.float32),
            jax.ShapeDtypeStruct((E_PAD, D_OUT), jnp.float32),
            jax.ShapeDtypeStruct((8, 128), jnp.float32),
        ],
        interpret=interpret,
    )(posj, spos16, xwj, wn_W, wn_b, W1, ln_b)


def _p2_body(np_ref, w_ref, st_ref, gnw_ref, gnb_ref,
             lng_ref, lnbe_ref, wng_ref, wnbe_ref, h_ref, st2_ref):
    i = pl.program_id(0)
    bm = h_ref.shape[0]
    inv_e = 1.0 / E
    ws = st_ref[0:1, 0:C_MID] * inv_e
    wq = st_ref[1:2, 0:C_MID] * inv_e
    ns = st_ref[2:3, :] * inv_e
    nq = st_ref[3:4, :] * inv_e
    wsc = wng_ref[...] / jnp.sqrt(wq - ws * ws + EPS)
    wsh = wnbe_ref[...] - ws * wsc
    nsc = lng_ref[...] / jnp.sqrt(nq - ns * ns + EPS)
    nsh = lnbe_ref[...] - ns * nsc

    npv = np_ref[...] * nsc + nsh
    wv = w_ref[...] * wsc + wsh
    np3 = npv.reshape(bm, K, D_OUT)
    w3 = wv.reshape(bm, K, C_MID)
    parts = []
    for c in range(C_MID):
        parts.append(jnp.sum(np3 * w3[:, :, c:c + 1], axis=1))
    agg = jnp.concatenate(parts, axis=1)
    h = jnp.maximum(
        jnp.dot(agg, gnw_ref[...], preferred_element_type=jnp.float32)
        + gnb_ref[...], 0.0)
    h_ref[...] = h

    midx = i * bm + lax.broadcasted_iota(jnp.int32, (bm, 1), 0)
    hm = jnp.where(midx < M, h, 0.0)

    @pl.when(i == 0)
    def _():
        st2_ref[...] = jnp.zeros_like(st2_ref)

    st2_ref[0:1, :] += jnp.sum(hm, axis=0, keepdims=True)
    st2_ref[1:2, :] += jnp.sum(hm * hm, axis=0, keepdims=True)


def _p2_call(np_pre, w_pre, st, gn_W2, gn_b, ln_g, ln_be, wn_g, wn_be,
             interpret=False):
    bm = 32
    be = bm * K
    grid = M_PAD // bm
    return pl.pallas_call(
        _p2_body,
        grid=(grid,),
        in_specs=[
            pl.BlockSpec((be, D_OUT), lambda i: (i, 0)),
            pl.BlockSpec((be, C_MID), lambda i: (i, 0)),
            pl.BlockSpec((8, 128), lambda i: (0, 0)),
            pl.BlockSpec((D_OUT * C_MID, D_GLOBAL), lambda i: (0, 0)),
            pl.BlockSpec((1, D_GLOBAL), lambda i: (0, 0)),
            pl.BlockSpec((1, D_OUT), lambda i: (0, 0)),
            pl.BlockSpec((1, D_OUT), lambda i: (0, 0)),
            pl.BlockSpec((1, C_MID), lambda i: (0, 0)),
            pl.BlockSpec((1, C_MID), lambda i: (0, 0)),
        ],
        out_specs=[
            pl.BlockSpec((bm, D_GLOBAL), lambda i: (i, 0)),
            pl.BlockSpec((8, D_GLOBAL), lambda i: (0, 0)),
        ],
        out_shape=[
            jax.ShapeDtypeStruct((M_PAD, D_GLOBAL), jnp.float32),
            jax.ShapeDtypeStruct((8, D_GLOBAL), jnp.float32),
        ],
        interpret=interpret,
    )(np_pre, w_pre, st, gn_W2, gn_b, ln_g, ln_be, wn_g, wn_be)


def _p3_body(h_ref, st2_ref, g_ref, be_ref, o_ref):
    inv_m = 1.0 / M
    mu = st2_ref[0:1, :] * inv_m
    var = st2_ref[1:2, :] * inv_m - mu * mu
    sc = g_ref[...] / jnp.sqrt(var + EPS)
    sh = be_ref[...] - mu * sc
    o_ref[...] = h_ref[...] * sc + sh


def _p3_call(h, st2, gn_g, gn_be, interpret=False):
    return pl.pallas_call(
        _p3_body,
        out_shape=jax.ShapeDtypeStruct((M_PAD, D_GLOBAL), jnp.float32),
        interpret=interpret,
    )(h, st2, gn_g, gn_be)


def kernel(pos, x, batch, wn_W, wn_b, wn_g, wn_be,
           ln_W, ln_b, ln_g, ln_be, gn_W, gn_b, gn_g, gn_be):
    posf = jnp.pad(pos, ((0, NROW * 128 - N), (0, 0)))
    px = posf[:, 0].reshape(NROW, 128)
    py = posf[:, 1].reshape(NROW, 128)
    pz = posf[:, 2].reshape(NROW, 128)
    pos_cols = jnp.pad(pos.T, ((0, 13), (0, NCOL - N)))
    pos16 = jnp.pad(pos, ((0, 0), (0, 13)))

    fps_idx = _fps_call(px, py, pz,
                        posf[:, 0], posf[:, 1], posf[:, 2])
    fps_sorted = jnp.sort(fps_idx)
    fps_pad = jnp.pad(fps_sorted, (0, M_PAD - M))

    xw = _xw_call(x, ln_W[3:])
    cat_table = jnp.concatenate(
        [xw, pos16, jnp.zeros((N, 112), jnp.float32)], axis=1)

    spos16 = _sc_gather(cat_table, fps_pad, chunk=80)[:, D_OUT:D_OUT + 16]

    nbr = _knn_call(spos16, pos_cols)
    nbr_flat = nbr.reshape(E_PAD)

    rows = _sc_gather(cat_table, nbr_flat, chunk=256)
    xwj = rows[:, :D_OUT]
    posj16 = rows[:, D_OUT:D_OUT + 16]

    w_pre, np_pre, st = _p1_call(posj16, spos16, xwj,
                                 wn_W, wn_b.reshape(1, -1),
                                 ln_W[:3], ln_b.reshape(1, -1))

    gn_W2 = gn_W.reshape(D_OUT, C_MID, D_GLOBAL).transpose(1, 0, 2) \
                .reshape(D_OUT * C_MID, D_GLOBAL)
    h, st2 = _p2_call(np_pre, w_pre, st, gn_W2,
                      gn_b.reshape(1, -1), ln_g.reshape(1, -1),
                      ln_be.reshape(1, -1), wn_g.reshape(1, -1),
                      wn_be.reshape(1, -1))

    out = _p3_call(h, st2, gn_g.reshape(1, -1), gn_be.reshape(1, -1))
    return out[:M]


# speedup vs baseline: 1.3923x; 1.3923x over previous
import functools

import jax
import jax.numpy as jnp
from jax import lax
from jax.experimental import pallas as pl
from jax.experimental.pallas import tpu as pltpu
from jax.experimental.pallas import tpu_sc as plsc

N = 10000
M = 2500
M_PAD = 2560
K = 32
C_MID = 16
D = 128
D_OUT = 128
D_GLOBAL = 256
EPS = 1e-5

NROW = 80
NCOL = 10112
E = M * K
E_PAD = M_PAD * K
BIG_I = 2**30


def _fps_body(px_ref, py_ref, pz_ref, psx_ref, psy_ref, psz_ref, idx_ref):
    px = px_ref[...]
    py = py_ref[...]
    pz = pz_ref[...]
    flat = (lax.broadcasted_iota(jnp.int32, (NROW, 128), 0) * 128
            + lax.broadcasted_iota(jnp.int32, (NROW, 128), 1))
    valid = flat < N
    lane = lax.broadcasted_iota(jnp.int32, (NROW, 128), 1)
    rowbase = lax.broadcasted_iota(jnp.int32, (NROW, 1), 0) * 128

    def dist_to(j):
        lx = psx_ref[j]
        ly = psy_ref[j]
        lz = psz_ref[j]
        return (px - lx) ** 2 + (py - ly) ** 2 + (pz - lz) ** 2

    idx_ref[0] = jnp.int32(0)
    dmin0 = jnp.where(valid, dist_to(jnp.int32(0)), -jnp.inf)

    def body(i, dmin):
        cmax = jnp.max(dmin, axis=1, keepdims=True)
        ccol = jnp.min(jnp.where(dmin == cmax, lane, BIG_I),
                       axis=1, keepdims=True)
        pc = rowbase + ccol
        mx = jnp.max(cmax)
        j = jnp.min(jnp.where(cmax == mx, pc, BIG_I))
        idx_ref[i] = j
        return jnp.minimum(dmin, dist_to(j))

    lax.fori_loop(1, M, body, dmin0)


def _fps_call(px, py, pz, psx, psy, psz, interpret=False):
    return pl.pallas_call(
        _fps_body,
        out_shape=jax.ShapeDtypeStruct((M,), jnp.int32),
        in_specs=[pl.BlockSpec(memory_space=pltpu.VMEM)] * 3
        + [pl.BlockSpec(memory_space=pltpu.SMEM)] * 3,
        out_specs=pl.BlockSpec(memory_space=pltpu.SMEM),
        interpret=interpret,
    )(px, py, pz, psx, psy, psz)


KNN_T = 8
KNN_G = NCOL // 128


def _knn_body(spos_ref, pc_ref, nbr_ref):
    bm = spos_ref.shape[0]
    sx = spos_ref[:, 0:1]
    sy = spos_ref[:, 1:2]
    sz = spos_ref[:, 2:3]
    px = pc_ref[0:1, :]
    py = pc_ref[1:2, :]
    pz = pc_ref[2:3, :]
    d2 = (sx - px) ** 2 + (sy - py) ** 2 + (sz - pz) ** 2
    cidx = lax.broadcasted_iota(jnp.int32, d2.shape, 1)
    d2 = jnp.where(cidx < N, d2, jnp.inf)

    inf = jnp.inf
    tvals = [jnp.full((bm, 128), inf, jnp.float32) for _ in range(KNN_T)]
    targs = [jnp.zeros((bm, 128), jnp.int32) for _ in range(KNN_T)]
    for g in range(KNN_G):
        cv = d2[:, g * 128:(g + 1) * 128]
        ca = None
        for p in range(KNN_T):
            sw = cv < tvals[p]
            tv_new = jnp.where(sw, cv, tvals[p])
            cv = jnp.where(sw, tvals[p], cv)
            if ca is None:
                ta_new = jnp.where(sw, g, targs[p])
                ca = jnp.where(sw, targs[p], g)
            else:
                ta_new = jnp.where(sw, ca, targs[p])
                ca = jnp.where(sw, targs[p], ca)
            tvals[p] = tv_new
            targs[p] = ta_new

    lidx = lax.broadcasted_iota(jnp.int32, (bm, 128), 1)
    cur, curg = tvals[0], targs[0]
    cnt = jnp.zeros((bm, 128), jnp.int32)
    cols = []
    for _ in range(K):
        v = jnp.min(cur, axis=1, keepdims=True)
        l = jnp.min(jnp.where(cur == v, lidx, BIG_I), axis=1, keepdims=True)
        oh = lidx == l
        g = jnp.sum(jnp.where(oh, curg, 0), axis=1, keepdims=True)
        cols.append(g * 128 + l)
        newc = cnt + 1
        nv = jnp.full((bm, 128), inf, jnp.float32)
        ng = jnp.zeros((bm, 128), jnp.int32)
        for p in range(1, KNN_T):
            hit = newc == p
            nv = jnp.where(hit, tvals[p], nv)
            ng = jnp.where(hit, targs[p], ng)
        cur = jnp.where(oh, nv, cur)
        curg = jnp.where(oh, ng, curg)
        cnt = jnp.where(oh, newc, cnt)
    nbr_ref[...] = jnp.concatenate(cols, axis=1)


def _knn_call(spos16, pos_cols, interpret=False):
    bm = 128
    grid = M_PAD // bm
    return pl.pallas_call(
        _knn_body,
        grid=(grid,),
        in_specs=[
            pl.BlockSpec((bm, 16), lambda i: (i, 0)),
            pl.BlockSpec((16, NCOL), lambda i: (0, 0)),
        ],
        out_specs=pl.BlockSpec((bm, K), lambda i: (i, 0)),
        out_shape=jax.ShapeDtypeStruct((M_PAD, K), jnp.int32),
        interpret=interpret,
    )(spos16, pos_cols)


def _xw_body(x_ref, w_ref, o_ref):
    o_ref[...] = jnp.dot(x_ref[...], w_ref[...],
                         preferred_element_type=jnp.float32)


def _xw_call(x, w2, interpret=False):
    bm = 1000
    return pl.pallas_call(
        _xw_body,
        grid=(N // bm,),
        in_specs=[
            pl.BlockSpec((bm, D), lambda i: (i, 0)),
            pl.BlockSpec((D, D_OUT), lambda i: (0, 0)),
        ],
        out_specs=pl.BlockSpec((bm, D_OUT), lambda i: (i, 0)),
        out_shape=jax.ShapeDtypeStruct((N, D_OUT), jnp.float32),
        interpret=interpret,
    )(x, w2)


def _sc_gather(table, idx, chunk):
    b = idx.shape[0]
    d = table.shape[1]
    nw = 32
    bpw = b // nw
    nch = bpw // chunk
    mesh = plsc.VectorSubcoreMesh(core_axis_name="c", subcore_axis_name="s")

    @functools.partial(
        pl.kernel,
        mesh=mesh,
        out_type=jax.ShapeDtypeStruct((b, d), jnp.float32),
        scratch_types=[
            pltpu.VMEM((chunk,), jnp.int32),
            pltpu.VMEM((chunk, d), jnp.float32),
            pltpu.SemaphoreType.DMA,
        ],
    )
    def k(table_hbm, idx_hbm, out_hbm, idx_v, rows_v, sem):
        wid = lax.axis_index("s") * 2 + lax.axis_index("c")
        base = wid * bpw
        for c in range(nch):
            off = base + c * chunk
            pltpu.sync_copy(idx_hbm.at[pl.ds(off, chunk)], idx_v)
            pltpu.async_copy(table_hbm.at[idx_v], rows_v, sem).wait()
            pltpu.sync_copy(rows_v, out_hbm.at[pl.ds(off, chunk)])

    return k(table, idx)


def _p1_body(posj_ref, spos_ref, xwj_ref, wnw_ref, wnb_ref, w1_ref, lnb_ref,
             wpre_ref, nppre_ref, st_ref):
    i = pl.program_id(0)
    bm = spos_ref.shape[0]
    be = bm * K
    spos = spos_ref[...]
    spos_e = jnp.broadcast_to(spos[:, None, :], (bm, K, 16)).reshape(be, 16)
    gn3 = (posj_ref[...] - spos_e)[:, 0:3]
    w = jnp.maximum(
        jnp.dot(gn3, wnw_ref[...], preferred_element_type=jnp.float32)
        + wnb_ref[...], 0.0)
    npv = jnp.maximum(
        jnp.dot(gn3, w1_ref[...], preferred_element_type=jnp.float32)
        + xwj_ref[...] + lnb_ref[...], 0.0)
    wpre_ref[...] = w
    nppre_ref[...] = npv

    eidx = i * be + lax.broadcasted_iota(jnp.int32, (be, 1), 0)
    mask = eidx < E
    wm = jnp.where(mask, w, 0.0)
    nm = jnp.where(mask, npv, 0.0)

    @pl.when(i == 0)
    def _():
        st_ref[...] = jnp.zeros_like(st_ref)

    st_ref[0:1, 0:C_MID] += jnp.sum(wm, axis=0, keepdims=True)
    st_ref[1:2, 0:C_MID] += jnp.sum(wm * wm, axis=0, keepdims=True)
    st_ref[2:3, :] += jnp.sum(nm, axis=0, keepdims=True)
    st_ref[3:4, :] += jnp.sum(nm * nm, axis=0, keepdims=True)


def _p1_call(posj, spos16, xwj, wn_W, wn_b, W1, ln_b, interpret=False):
    bm = 32
    be = bm * K
    grid = M_PAD // bm
    return pl.pallas_call(
        _p1_body,
        grid=(grid,),
        in_specs=[
            pl.BlockSpec((be, 16), lambda i: (i, 0)),
            pl.BlockSpec((bm, 16), lambda i: (i, 0)),
            pl.BlockSpec((be, D_OUT), lambda i: (i, 0)),
            pl.BlockSpec((3, C_MID), lambda i: (0, 0)),
            pl.BlockSpec((1, C_MID), lambda i: (0, 0)),
            pl.BlockSpec((3, D_OUT), lambda i: (0, 0)),
            pl.BlockSpec((1, D_OUT), lambda i: (0, 0)),
        ],
        out_specs=[
            pl.BlockSpec((be, C_MID), lambda i: (i, 0)),
            pl.BlockSpec((be, D_OUT), lambda i: (i, 0)),
            pl.BlockSpec((8, 128), lambda i: (0, 0)),
        ],
        out_shape=[
            jax.ShapeDtypeStruct((E_PAD, C_MID), jnp.float32),
            jax.ShapeDtypeStruct((E_PAD, D_OUT), jnp.float32),
            jax.ShapeDtypeStruct((8, 128), jnp.float32),
        ],
        interpret=interpret,
    )(posj, spos16, xwj, wn_W, wn_b, W1, ln_b)


def _p2_body(np_ref, w_ref, st_ref, gnw_ref, gnb_ref,
             lng_ref, lnbe_ref, wng_ref, wnbe_ref, h_ref, st2_ref):
    i = pl.program_id(0)
    bm = h_ref.shape[0]
    inv_e = 1.0 / E
    ws = st_ref[0:1, 0:C_MID] * inv_e
    wq = st_ref[1:2, 0:C_MID] * inv_e
    ns = st_ref[2:3, :] * inv_e
    nq = st_ref[3:4, :] * inv_e
    wsc = wng_ref[...] / jnp.sqrt(wq - ws * ws + EPS)
    wsh = wnbe_ref[...] - ws * wsc
    nsc = lng_ref[...] / jnp.sqrt(nq - ns * ns + EPS)
    nsh = lnbe_ref[...] - ns * nsc

    npv = np_ref[...] * nsc + nsh
    wv = w_ref[...] * wsc + wsh
    np3 = npv.reshape(bm, K, D_OUT)
    w3 = wv.reshape(bm, K, C_MID)
    parts = []
    for c in range(C_MID):
        parts.append(jnp.sum(np3 * w3[:, :, c:c + 1], axis=1))
    agg = jnp.concatenate(parts, axis=1)
    h = jnp.maximum(
        jnp.dot(agg, gnw_ref[...], preferred_element_type=jnp.float32)
        + gnb_ref[...], 0.0)
    h_ref[...] = h

    midx = i * bm + lax.broadcasted_iota(jnp.int32, (bm, 1), 0)
    hm = jnp.where(midx < M, h, 0.0)

    @pl.when(i == 0)
    def _():
        st2_ref[...] = jnp.zeros_like(st2_ref)

    st2_ref[0:1, :] += jnp.sum(hm, axis=0, keepdims=True)
    st2_ref[1:2, :] += jnp.sum(hm * hm, axis=0, keepdims=True)


def _p2_call(np_pre, w_pre, st, gn_W2, gn_b, ln_g, ln_be, wn_g, wn_be,
             interpret=False):
    bm = 32
    be = bm * K
    grid = M_PAD // bm
    return pl.pallas_call(
        _p2_body,
        grid=(grid,),
        in_specs=[
            pl.BlockSpec((be, D_OUT), lambda i: (i, 0)),
            pl.BlockSpec((be, C_MID), lambda i: (i, 0)),
            pl.BlockSpec((8, 128), lambda i: (0, 0)),
            pl.BlockSpec((D_OUT * C_MID, D_GLOBAL), lambda i: (0, 0)),
            pl.BlockSpec((1, D_GLOBAL), lambda i: (0, 0)),
            pl.BlockSpec((1, D_OUT), lambda i: (0, 0)),
            pl.BlockSpec((1, D_OUT), lambda i: (0, 0)),
            pl.BlockSpec((1, C_MID), lambda i: (0, 0)),
            pl.BlockSpec((1, C_MID), lambda i: (0, 0)),
        ],
        out_specs=[
            pl.BlockSpec((bm, D_GLOBAL), lambda i: (i, 0)),
            pl.BlockSpec((8, D_GLOBAL), lambda i: (0, 0)),
        ],
        out_shape=[
            jax.ShapeDtypeStruct((M_PAD, D_GLOBAL), jnp.float32),
            jax.ShapeDtypeStruct((8, D_GLOBAL), jnp.float32),
        ],
        interpret=interpret,
    )(np_pre, w_pre, st, gn_W2, gn_b, ln_g, ln_be, wn_g, wn_be)


def _p3_body(h_ref, st2_ref, g_ref, be_ref, o_ref):
    inv_m = 1.0 / M
    mu = st2_ref[0:1, :] * inv_m
    var = st2_ref[1:2, :] * inv_m - mu * mu
    sc = g_ref[...] / jnp.sqrt(var + EPS)
    sh = be_ref[...] - mu * sc
    o_ref[...] = h_ref[...] * sc + sh


def _p3_call(h, st2, gn_g, gn_be, interpret=False):
    return pl.pallas_call(
        _p3_body,
        out_shape=jax.ShapeDtypeStruct((M_PAD, D_GLOBAL), jnp.float32),
        interpret=interpret,
    )(h, st2, gn_g, gn_be)


def kernel(pos, x, batch, wn_W, wn_b, wn_g, wn_be,
           ln_W, ln_b, ln_g, ln_be, gn_W, gn_b, gn_g, gn_be):
    posf = jnp.pad(pos, ((0, NROW * 128 - N), (0, 0)))
    px = posf[:, 0].reshape(NROW, 128)
    py = posf[:, 1].reshape(NROW, 128)
    pz = posf[:, 2].reshape(NROW, 128)
    pos_cols = jnp.pad(pos.T, ((0, 13), (0, NCOL - N)))
    pos16 = jnp.pad(pos, ((0, 0), (0, 13)))

    fps_idx = _fps_call(px, py, pz,
                        posf[:, 0], posf[:, 1], posf[:, 2])
    fps_sorted = jnp.sort(fps_idx)
    fps_pad = jnp.pad(fps_sorted, (0, M_PAD - M))

    xw = _xw_call(x, ln_W[3:])
    cat_table = jnp.concatenate(
        [xw, pos16, jnp.zeros((N, 112), jnp.float32)], axis=1)

    spos16 = _sc_gather(cat_table, fps_pad, chunk=80)[:, D_OUT:D_OUT + 16]

    nbr = _knn_call(spos16, pos_cols)
    nbr_flat = nbr.reshape(E_PAD)

    rows = _sc_gather(cat_table, nbr_flat, chunk=256)
    xwj = rows[:, :D_OUT]
    posj16 = rows[:, D_OUT:D_OUT + 16]

    w_pre, np_pre, st = _p1_call(posj16, spos16, xwj,
                                 wn_W, wn_b.reshape(1, -1),
                                 ln_W[:3], ln_b.reshape(1, -1))

    gn_W2 = gn_W.reshape(D_OUT, C_MID, D_GLOBAL).transpose(1, 0, 2) \
                .reshape(D_OUT * C_MID, D_GLOBAL)
    h, st2 = _p2_call(np_pre, w_pre, st, gn_W2,
                      gn_b.reshape(1, -1), ln_g.reshape(1, -1),
                      ln_be.reshape(1, -1), wn_g.reshape(1, -1),
                      wn_be.reshape(1, -1))

    out = _p3_call(h, st2, gn_g.reshape(1, -1), gn_be.reshape(1, -1))
    return out[:M]
